# baseline (speedup 1.0000x reference)
import functools
import math

import numpy as np
import jax
import jax.numpy as jnp
from jax import lax
from jax.experimental import pallas as pl
from jax.experimental.pallas import tpu as pltpu
from jax.experimental.pallas import tpu_sc as plsc

K = 10
N = 10000
E = 320000
F = 128
NC, NS, L = 2, 16, 16
NW = NC * NS
EPW = E // NW
CH = 80
NCH = EPW // CH
NRCH = N // CH
RK = -(-NRCH // NS)
EPT = E // NS
WCH = 2000

_mesh = plsc.VectorSubcoreMesh(core_axis_name="c", subcore_axis_name="s")


def _bernstein_monomial_matrix():
    C = np.zeros((K + 1, K + 1), dtype=np.float64)
    for j in range(K + 1):
        for p in range(j + 1):
            for q in range(K - j + 1):
                C[p + q, j] += (
                    math.comb(j, p) * ((-1) ** p) * math.comb(K - j, q) * math.comb(K, j)
                )
    return C / float(2 ** K)


_CMAT = _bernstein_monomial_matrix().astype(np.float32)


@functools.partial(
    pl.kernel,
    out_type=jax.ShapeDtypeStruct((E,), jnp.float32),
    mesh=_mesh,
    compiler_params=pltpu.CompilerParams(needs_layout_passes=False),
    scratch_types=[
        pltpu.VMEM((N,), jnp.float32),
        pltpu.VMEM((WCH,), jnp.int32),
        pltpu.VMEM((WCH,), jnp.int32),
        pltpu.VMEM((WCH,), jnp.float32),
        pltpu.VMEM((WCH,), jnp.float32),
        pltpu.VMEM((N,), jnp.float32),
        pltpu.VMEM((N,), jnp.float32),
        pltpu.VMEM((N,), jnp.float32),
        pltpu.VMEM_SHARED((NS, N), jnp.float32),
        pltpu.VMEM_SHARED((N,), jnp.float32),
    ],
)
def _norm_kernel(rows_hbm, cols_hbm, vals_hbm, w_hbm,
                 degloc, rbuf, cbuf, vbuf, wchunk, acc, tmp, disloc,
                 deg_all, dis_sh):
    c = lax.axis_index("c")
    t = lax.axis_index("s")

    @pl.when(c == 0)
    def _core0():
        def zloc(i, _):
            degloc[pl.ds(i * L, L)] = jnp.zeros((L,), jnp.float32)
            return 0
        lax.fori_loop(0, N // L, zloc, 0)

        base = t * EPT

        def dchunk(i, _):
            off = base + i * WCH
            pltpu.sync_copy(rows_hbm.at[pl.ds(off, WCH)], rbuf)
            pltpu.sync_copy(vals_hbm.at[pl.ds(off, WCH)], vbuf)

            def inner(k2, _2):
                r16 = rbuf[pl.ds(k2 * L, L)]
                v16 = vbuf[pl.ds(k2 * L, L)]
                plsc.addupdate_scatter(degloc, [r16], v16)
                return 0
            lax.fori_loop(0, WCH // L, inner, 0)
            return 0
        lax.fori_loop(0, EPT // WCH, dchunk, 0)

        pltpu.sync_copy(degloc, deg_all.at[t])
        plsc.subcore_barrier()

        @pl.when(t == 0)
        def _reduce():
            def zacc(i, _):
                acc[pl.ds(i * L, L)] = jnp.zeros((L,), jnp.float32)
                return 0
            lax.fori_loop(0, N // L, zacc, 0)

            def red(j, _):
                pltpu.sync_copy(deg_all.at[j], tmp)

                def addv(i, _2):
                    acc[pl.ds(i * L, L)] = acc[pl.ds(i * L, L)] + tmp[pl.ds(i * L, L)]
                    return 0
                lax.fori_loop(0, N // L, addv, 0)
                return 0
            lax.fori_loop(0, NS, red, 0)

            def rsq(i, _):
                d = acc[pl.ds(i * L, L)]
                bi = plsc.bitcast(d, jnp.int32)
                y = plsc.bitcast(jnp.int32(0x5F3759DF) - (bi >> 1), jnp.float32)
                for _it in range(4):
                    y = y * (1.5 - 0.5 * d * y * y)
                y = jnp.where(d > 0.0, y, jnp.zeros((L,), jnp.float32))
                acc[pl.ds(i * L, L)] = y
                return 0
            lax.fori_loop(0, N // L, rsq, 0)
            pltpu.sync_copy(acc, dis_sh)

        plsc.subcore_barrier()
        pltpu.sync_copy(dis_sh, disloc)

        def wloop(i, _):
            off = base + i * WCH
            pltpu.sync_copy(rows_hbm.at[pl.ds(off, WCH)], rbuf)
            pltpu.sync_copy(cols_hbm.at[pl.ds(off, WCH)], cbuf)
            pltpu.sync_copy(vals_hbm.at[pl.ds(off, WCH)], vbuf)

            def winner(k2, _2):
                r16 = rbuf[pl.ds(k2 * L, L)]
                c16 = cbuf[pl.ds(k2 * L, L)]
                dr = plsc.load_gather(disloc, [r16])
                dc = plsc.load_gather(disloc, [c16])
                wchunk[pl.ds(k2 * L, L)] = vbuf[pl.ds(k2 * L, L)] * dr * dc
                return 0
            lax.fori_loop(0, WCH // L, winner, 0)
            pltpu.sync_copy(wchunk, w_hbm.at[pl.ds(off, WCH)])
            return 0
        lax.fori_loop(0, EPT // WCH, wloop, 0)


RCH = 80


def _make_spmm(use_shuf, ch, epw):
    nch = epw // ch

    scratch = [
        pltpu.VMEM_SHARED((N, F), jnp.float32),
        pltpu.VMEM((ch, F), jnp.float32),
        pltpu.VMEM((ch, F), jnp.float32),
        pltpu.VMEM((ch,), jnp.int32),
        pltpu.VMEM((ch,), jnp.int32),
        pltpu.VMEM((ch,), jnp.float32),
        pltpu.VMEM((ch,), jnp.float32),
        pltpu.VMEM((epw,), jnp.int32),
        pltpu.VMEM((L,), jnp.float32),
    ]
    if use_shuf:
        scratch += [
            pltpu.VMEM((ch,), jnp.int32),
            pltpu.VMEM((ch,), jnp.int32),
            pltpu.VMEM((N,), jnp.int32),
        ]
    scratch += [
        pltpu.SemaphoreType.DMA,
        pltpu.SemaphoreType.DMA,
        pltpu.SemaphoreType.DMA,
        pltpu.SemaphoreType.DMA,
    ]

    @functools.partial(
        pl.kernel,
        out_type=jax.ShapeDtypeStruct((NC * N, F), jnp.float32),
        mesh=_mesh,
        compiler_params=pltpu.CompilerParams(needs_layout_passes=False),
        scratch_types=scratch,
    )
    def _spmm(v_hbm, rows_hbm, cols_hbm, w_hbm, s_hbm, shuf_hbm, u_hbm, *scr):
        if use_shuf:
            (u_sh, gbufA, gbufB, rowA, rowB, wA, wB, colall, sbuf,
             gidxA, gidxB, shufall, semGA, semGB, semEA, semEB) = scr
        else:
            (u_sh, gbufA, gbufB, rowA, rowB, wA, wB, colall, sbuf,
             semGA, semGB, semEA, semEB) = scr
            gidxA = gidxB = shufall = None

        c = lax.axis_index("c")
        t = lax.axis_index("s")
        wid = c * NS + t
        base = wid * epw

        gsta = gbufA.at[pl.ds(0, RCH)]

        def zb(i, _):
            for d in range(F // L):
                gbufA[i, pl.ds(d * L, L)] = jnp.zeros((L,), jnp.float32)
            return 0
        lax.fori_loop(0, RCH, zb, 0)
        for k in range(RK):
            idx = t + k * NS

            @pl.when(idx < NRCH)
            def _zchunk():
                pltpu.sync_copy(gsta, u_sh.at[pl.ds(idx * RCH, RCH)])

        pltpu.sync_copy(cols_hbm.at[pl.ds(base, epw)], colall)
        pltpu.sync_copy(s_hbm, sbuf)
        if use_shuf:
            pltpu.sync_copy(shuf_hbm, shufall)
        plsc.subcore_barrier()

        def gsrc(off, gidx):
            if use_shuf:
                return v_hbm.at[gidx]
            return v_hbm.at[colall.at[pl.ds(off, ch)]]

        def start(i, gbuf, rowbuf, wbuf, gidx, semG, semE):
            off = i * ch
            pltpu.async_copy(rows_hbm.at[pl.ds(base + off, ch)], rowbuf, semE)
            pltpu.async_copy(w_hbm.at[pl.ds(base + off, ch)], wbuf, semE)
            if use_shuf:
                for k in range(ch // L):
                    c16 = colall[pl.ds(off + k * L, L)]
                    gidx[pl.ds(k * L, L)] = plsc.load_gather(shufall, [c16])
            pltpu.async_copy(gsrc(off, gidx), gbuf, semG)

        def process(i, gbuf, rowbuf, wbuf, gidx, semG, semE):
            off = i * ch
            pltpu.make_async_copy(rows_hbm.at[pl.ds(base + off, ch)], rowbuf, semE).wait()
            pltpu.make_async_copy(w_hbm.at[pl.ds(base + off, ch)], wbuf, semE).wait()
            pltpu.make_async_copy(gsrc(off, gidx), gbuf, semG).wait()
            sv = sbuf[...]

            def scale(g, _2):
                w16 = wbuf[pl.ds(g * L, L)] * sv
                for j in range(L):
                    e = g * L + j
                    ws = w16[j]
                    for d in range(F // L):
                        gbuf[e, pl.ds(d * L, L)] = gbuf[e, pl.ds(d * L, L)] * ws
                return 0
            lax.fori_loop(0, ch // L, scale, 0)
            pltpu.sync_copy(gbuf, u_sh.at[rowbuf], add=True)

        start(0, gbufA, rowA, wA, gidxA, semGA, semEA)

        def pair(i2, _):
            i = i2 * 2
            start(i + 1, gbufB, rowB, wB, gidxB, semGB, semEB)
            process(i, gbufA, rowA, wA, gidxA, semGA, semEA)

            @pl.when(i + 2 < nch)
            def _startA():
                start(i + 2, gbufA, rowA, wA, gidxA, semGA, semEA)
            process(i + 1, gbufB, rowB, wB, gidxB, semGB, semEB)
            return 0
        lax.fori_loop(0, nch // 2, pair, 0)
        if nch % 2 == 1:
            process(nch - 1, gbufA, rowA, wA, gidxA, semGA, semEA)
        plsc.subcore_barrier()

        for k in range(RK):
            idx = t + k * NS

            @pl.when(idx < NRCH)
            def _dchunk():
                r0 = idx * RCH
                pltpu.sync_copy(u_sh.at[pl.ds(r0, RCH)], gsta)
                pltpu.sync_copy(gsta, u_hbm.at[pl.ds(c * N + r0, RCH)])

    return _spmm


CHP = 128
EPWP = -(-EPW // CHP) * CHP
EP2 = NW * EPWP
_spmm_plain = _make_spmm(False, CHP, EPWP)
_spmm_shuf = _make_spmm(True, CH, EPW)


def _comb_body(a_ref, u0_ref, u1_ref, x_ref, y_ref):
    y_ref[...] = u0_ref[...] + u1_ref[...] + a_ref[0, 0] * x_ref[...]


_comb_call = pl.pallas_call(
    _comb_body,
    grid=(10,),
    in_specs=[
        pl.BlockSpec(memory_space=pltpu.SMEM),
        pl.BlockSpec((N // 10, F), lambda i: (i, 0)),
        pl.BlockSpec((N // 10, F), lambda i: (i, 0)),
        pl.BlockSpec((N // 10, F), lambda i: (i, 0)),
    ],
    out_specs=pl.BlockSpec((N // 10, F), lambda i: (i, 0)),
    out_shape=jax.ShapeDtypeStruct((N, F), jnp.float32),
)


def _comb(alpha, U, x):
    return _comb_call(alpha.reshape(1, 1), U[:N], U[N:], x)


def _pad_edges(r, cc, ww):
    pad = EP2 - E
    rp = jnp.concatenate([r, jnp.zeros((pad,), jnp.int32)])
    cp = jnp.concatenate([cc, jnp.zeros((pad,), jnp.int32)])
    wp = jnp.concatenate([ww, jnp.zeros((pad,), jnp.float32)])
    return rp, cp, wp


def kernel(x, shuf, adj_indices, adj_values, neighbor_indices, neighbor_values, temp):
    TEMP = jax.nn.relu(temp)
    a = jnp.asarray(_CMAT) @ TEMP

    arow, acol = adj_indices[0], adj_indices[1]
    nrow, ncol = neighbor_indices[0], neighbor_indices[1]

    w = _norm_kernel(arow, acol, adj_values)

    arow_p, acol_p, w_p = _pad_edges(arow, acol, w)
    nrow_p, ncol_p, nv_p = _pad_edges(nrow, ncol, neighbor_values)

    ones16 = jnp.ones((L,), jnp.float32)
    zero11 = jnp.zeros((1, 1), jnp.float32)

    U = _spmm_plain(x, arow_p, acol_p, w_p, jnp.full((L,), a[K], jnp.float32), shuf)
    y = _comb(a[K - 1], U, x)
    for m in range(K - 2, -1, -1):
        U = _spmm_plain(y, arow_p, acol_p, w_p, ones16, shuf)
        y = _comb(a[m], U, x)
    out = y

    U = _spmm_plain(out, nrow_p, ncol_p, nv_p, ones16, shuf)
    z1 = _comb(zero11, U, x)
    U = _spmm_plain(z1, nrow_p, ncol_p, nv_p, ones16, shuf)
    z_pos = _comb(zero11, U, x)

    U = _spmm_shuf(out, nrow, ncol, neighbor_values, ones16, shuf)
    s1 = _comb(zero11, U, x)
    U = _spmm_plain(s1, nrow_p, ncol_p, nv_p, ones16, shuf)
    z_neg = _comb(zero11, U, x)

    return (out, z_pos, z_neg)


# speedup vs baseline: 1.3119x; 1.3119x over previous
import functools
import math

import numpy as np
import jax
import jax.numpy as jnp
from jax import lax
from jax.experimental import pallas as pl
from jax.experimental.pallas import tpu as pltpu
from jax.experimental.pallas import tpu_sc as plsc

K = 10
N = 10000
E = 320000
F = 128
NC, NS, L = 2, 16, 16
NW = NC * NS
EPW = E // NW
CH = 80
NCH = EPW // CH
NRCH = N // CH
RK = -(-NRCH // NS)
EPT = E // NS
WCH = 2000

_mesh = plsc.VectorSubcoreMesh(core_axis_name="c", subcore_axis_name="s")


def _bernstein_monomial_matrix():
    C = np.zeros((K + 1, K + 1), dtype=np.float64)
    for j in range(K + 1):
        for p in range(j + 1):
            for q in range(K - j + 1):
                C[p + q, j] += (
                    math.comb(j, p) * ((-1) ** p) * math.comb(K - j, q) * math.comb(K, j)
                )
    return C / float(2 ** K)


_CMAT = _bernstein_monomial_matrix().astype(np.float32)


@functools.partial(
    pl.kernel,
    out_type=jax.ShapeDtypeStruct((E,), jnp.float32),
    mesh=_mesh,
    compiler_params=pltpu.CompilerParams(needs_layout_passes=False),
    scratch_types=[
        pltpu.VMEM((N,), jnp.float32),
        pltpu.VMEM((WCH,), jnp.int32),
        pltpu.VMEM((WCH,), jnp.int32),
        pltpu.VMEM((WCH,), jnp.float32),
        pltpu.VMEM((WCH,), jnp.float32),
        pltpu.VMEM((N,), jnp.float32),
        pltpu.VMEM((N,), jnp.float32),
        pltpu.VMEM((N,), jnp.float32),
        pltpu.VMEM_SHARED((NS, N), jnp.float32),
        pltpu.VMEM_SHARED((N,), jnp.float32),
    ],
)
def _norm_kernel(rows_hbm, cols_hbm, vals_hbm, w_hbm,
                 degloc, rbuf, cbuf, vbuf, wchunk, acc, tmp, disloc,
                 deg_all, dis_sh):
    c = lax.axis_index("c")
    t = lax.axis_index("s")

    @pl.when(c == 0)
    def _core0():
        def zloc(i, _):
            degloc[pl.ds(i * L, L)] = jnp.zeros((L,), jnp.float32)
            return 0
        lax.fori_loop(0, N // L, zloc, 0)

        base = t * EPT

        def dchunk(i, _):
            off = base + i * WCH
            pltpu.sync_copy(rows_hbm.at[pl.ds(off, WCH)], rbuf)
            pltpu.sync_copy(vals_hbm.at[pl.ds(off, WCH)], vbuf)

            def inner(k2, _2):
                r16 = rbuf[pl.ds(k2 * L, L)]
                v16 = vbuf[pl.ds(k2 * L, L)]
                plsc.addupdate_scatter(degloc, [r16], v16)
                return 0
            lax.fori_loop(0, WCH // L, inner, 0)
            return 0
        lax.fori_loop(0, EPT // WCH, dchunk, 0)

        pltpu.sync_copy(degloc, deg_all.at[t])
        plsc.subcore_barrier()

        @pl.when(t == 0)
        def _reduce():
            def zacc(i, _):
                acc[pl.ds(i * L, L)] = jnp.zeros((L,), jnp.float32)
                return 0
            lax.fori_loop(0, N // L, zacc, 0)

            def red(j, _):
                pltpu.sync_copy(deg_all.at[j], tmp)

                def addv(i, _2):
                    acc[pl.ds(i * L, L)] = acc[pl.ds(i * L, L)] + tmp[pl.ds(i * L, L)]
                    return 0
                lax.fori_loop(0, N // L, addv, 0)
                return 0
            lax.fori_loop(0, NS, red, 0)

            def rsq(i, _):
                d = acc[pl.ds(i * L, L)]
                bi = plsc.bitcast(d, jnp.int32)
                y = plsc.bitcast(jnp.int32(0x5F3759DF) - (bi >> 1), jnp.float32)
                for _it in range(4):
                    y = y * (1.5 - 0.5 * d * y * y)
                y = jnp.where(d > 0.0, y, jnp.zeros((L,), jnp.float32))
                acc[pl.ds(i * L, L)] = y
                return 0
            lax.fori_loop(0, N // L, rsq, 0)
            pltpu.sync_copy(acc, dis_sh)

        plsc.subcore_barrier()
        pltpu.sync_copy(dis_sh, disloc)

        def wloop(i, _):
            off = base + i * WCH
            pltpu.sync_copy(rows_hbm.at[pl.ds(off, WCH)], rbuf)
            pltpu.sync_copy(cols_hbm.at[pl.ds(off, WCH)], cbuf)
            pltpu.sync_copy(vals_hbm.at[pl.ds(off, WCH)], vbuf)

            def winner(k2, _2):
                r16 = rbuf[pl.ds(k2 * L, L)]
                c16 = cbuf[pl.ds(k2 * L, L)]
                dr = plsc.load_gather(disloc, [r16])
                dc = plsc.load_gather(disloc, [c16])
                wchunk[pl.ds(k2 * L, L)] = vbuf[pl.ds(k2 * L, L)] * dr * dc
                return 0
            lax.fori_loop(0, WCH // L, winner, 0)
            pltpu.sync_copy(wchunk, w_hbm.at[pl.ds(off, WCH)])
            return 0
        lax.fori_loop(0, EPT // WCH, wloop, 0)


RCH = 80


def _make_spmm(use_shuf, ch, epw):
    nch = epw // ch

    scratch = [
        pltpu.VMEM_SHARED((N, F), jnp.float32),
        pltpu.VMEM((ch, F), jnp.float32),
        pltpu.VMEM((ch, F), jnp.float32),
        pltpu.VMEM((ch,), jnp.int32),
        pltpu.VMEM((ch,), jnp.int32),
        pltpu.VMEM((ch,), jnp.float32),
        pltpu.VMEM((ch,), jnp.float32),
        pltpu.VMEM((epw,), jnp.int32),
        pltpu.VMEM((L,), jnp.float32),
    ]
    if use_shuf:
        scratch += [
            pltpu.VMEM((ch,), jnp.int32),
            pltpu.VMEM((ch,), jnp.int32),
            pltpu.VMEM((N,), jnp.int32),
        ]
    scratch += [
        pltpu.SemaphoreType.DMA,
        pltpu.SemaphoreType.DMA,
        pltpu.SemaphoreType.DMA,
        pltpu.SemaphoreType.DMA,
    ]

    @functools.partial(
        pl.kernel,
        out_type=jax.ShapeDtypeStruct((NC * N, F), jnp.float32),
        mesh=_mesh,
        compiler_params=pltpu.CompilerParams(needs_layout_passes=False),
        scratch_types=scratch,
    )
    def _spmm(v_hbm, rows_hbm, cols_hbm, w_hbm, s_hbm, shuf_hbm, u_hbm, *scr):
        if use_shuf:
            (u_sh, gbufA, gbufB, rowA, rowB, wA, wB, colall, sbuf,
             gidxA, gidxB, shufall, semGA, semGB, semEA, semEB) = scr
        else:
            (u_sh, gbufA, gbufB, rowA, rowB, wA, wB, colall, sbuf,
             semGA, semGB, semEA, semEB) = scr
            gidxA = gidxB = shufall = None

        c = lax.axis_index("c")
        t = lax.axis_index("s")
        wid = c * NS + t
        base = wid * epw

        gsta = gbufA.at[pl.ds(0, RCH)]

        def zb(i, _):
            for d in range(F // L):
                gbufA[i, pl.ds(d * L, L)] = jnp.zeros((L,), jnp.float32)
            return 0
        lax.fori_loop(0, RCH, zb, 0)
        for k in range(RK):
            idx = t + k * NS

            @pl.when(idx < NRCH)
            def _zchunk():
                pltpu.sync_copy(gsta, u_sh.at[pl.ds(idx * RCH, RCH)])

        pltpu.sync_copy(cols_hbm.at[pl.ds(base, epw)], colall)
        pltpu.sync_copy(s_hbm, sbuf)
        if use_shuf:
            pltpu.sync_copy(shuf_hbm, shufall)
        plsc.subcore_barrier()

        def gsrc(off, gidx):
            if use_shuf:
                return v_hbm.at[gidx]
            return v_hbm.at[colall.at[pl.ds(off, ch)]]

        def start(i, gbuf, rowbuf, wbuf, gidx, semG, semE):
            off = i * ch
            pltpu.async_copy(rows_hbm.at[pl.ds(base + off, ch)], rowbuf, semE)
            pltpu.async_copy(w_hbm.at[pl.ds(base + off, ch)], wbuf, semE)
            if use_shuf:
                for k in range(ch // L):
                    c16 = colall[pl.ds(off + k * L, L)]
                    gidx[pl.ds(k * L, L)] = plsc.load_gather(shufall, [c16])
            pltpu.async_copy(gsrc(off, gidx), gbuf, semG)

        def process(i, gbuf, rowbuf, wbuf, gidx, semG, semE):
            off = i * ch
            pltpu.make_async_copy(rows_hbm.at[pl.ds(base + off, ch)], rowbuf, semE).wait()
            pltpu.make_async_copy(w_hbm.at[pl.ds(base + off, ch)], wbuf, semE).wait()
            pltpu.make_async_copy(gsrc(off, gidx), gbuf, semG).wait()
            sv = sbuf[...]

            def scale(g, _2):
                w16 = wbuf[pl.ds(g * L, L)] * sv
                for j in range(L):
                    e = g * L + j
                    ws = w16[j]
                    for d in range(F // L):
                        gbuf[e, pl.ds(d * L, L)] = gbuf[e, pl.ds(d * L, L)] * ws
                return 0
            lax.fori_loop(0, ch // L, scale, 0)
            pltpu.sync_copy(gbuf, u_sh.at[rowbuf], add=True)

        start(0, gbufA, rowA, wA, gidxA, semGA, semEA)

        def pair(i2, _):
            i = i2 * 2
            start(i + 1, gbufB, rowB, wB, gidxB, semGB, semEB)
            process(i, gbufA, rowA, wA, gidxA, semGA, semEA)

            @pl.when(i + 2 < nch)
            def _startA():
                start(i + 2, gbufA, rowA, wA, gidxA, semGA, semEA)
            process(i + 1, gbufB, rowB, wB, gidxB, semGB, semEB)
            return 0
        lax.fori_loop(0, nch // 2, pair, 0)
        if nch % 2 == 1:
            process(nch - 1, gbufA, rowA, wA, gidxA, semGA, semEA)
        plsc.subcore_barrier()

        for k in range(RK):
            idx = t + k * NS

            @pl.when(idx < NRCH)
            def _dchunk():
                r0 = idx * RCH
                pltpu.sync_copy(u_sh.at[pl.ds(r0, RCH)], gsta)
                pltpu.sync_copy(gsta, u_hbm.at[pl.ds(c * N + r0, RCH)])

    return _spmm


CHP = 64
EPWP = -(-EPW // CHP) * CHP
EP2 = NW * EPWP
_spmm_plain = _make_spmm(False, CHP, EPWP)
_spmm_shuf = _make_spmm(True, CH, EPW)


def _comb_body(a_ref, u0_ref, u1_ref, x_ref, y_ref):
    y_ref[...] = u0_ref[...] + u1_ref[...] + a_ref[0, 0] * x_ref[...]


_comb_call = pl.pallas_call(
    _comb_body,
    grid=(10,),
    in_specs=[
        pl.BlockSpec(memory_space=pltpu.SMEM),
        pl.BlockSpec((N // 10, F), lambda i: (i, 0)),
        pl.BlockSpec((N // 10, F), lambda i: (i, 0)),
        pl.BlockSpec((N // 10, F), lambda i: (i, 0)),
    ],
    out_specs=pl.BlockSpec((N // 10, F), lambda i: (i, 0)),
    out_shape=jax.ShapeDtypeStruct((N, F), jnp.float32),
)


def _comb(alpha, U, x):
    return _comb_call(alpha.reshape(1, 1), U[:N], U[N:], x)


def _pad_edges(r, cc, ww):
    pad = EP2 - E
    rp = jnp.concatenate([r, jnp.zeros((pad,), jnp.int32)])
    cp = jnp.concatenate([cc, jnp.zeros((pad,), jnp.int32)])
    wp = jnp.concatenate([ww, jnp.zeros((pad,), jnp.float32)])
    return rp, cp, wp


def kernel(x, shuf, adj_indices, adj_values, neighbor_indices, neighbor_values, temp):
    TEMP = jax.nn.relu(temp)
    a = jnp.asarray(_CMAT) @ TEMP

    arow, acol = adj_indices[0], adj_indices[1]
    nrow, ncol = neighbor_indices[0], neighbor_indices[1]

    w = _norm_kernel(arow, acol, adj_values)

    arow_p, acol_p, w_p = _pad_edges(arow, acol, w)
    nrow_p, ncol_p, nv_p = _pad_edges(nrow, ncol, neighbor_values)

    ones16 = jnp.ones((L,), jnp.float32)
    zero11 = jnp.zeros((1, 1), jnp.float32)

    U = _spmm_plain(x, arow_p, acol_p, w_p, jnp.full((L,), a[K], jnp.float32), shuf)
    y = _comb(a[K - 1], U, x)
    for m in range(K - 2, -1, -1):
        U = _spmm_plain(y, arow_p, acol_p, w_p, ones16, shuf)
        y = _comb(a[m], U, x)
    out = y

    U = _spmm_plain(out, nrow_p, ncol_p, nv_p, ones16, shuf)
    z1 = _comb(zero11, U, x)
    U = _spmm_plain(z1, nrow_p, ncol_p, nv_p, ones16, shuf)
    z_pos = _comb(zero11, U, x)

    U = _spmm_shuf(out, nrow, ncol, neighbor_values, ones16, shuf)
    s1 = _comb(zero11, U, x)
    U = _spmm_plain(s1, nrow_p, ncol_p, nv_p, ones16, shuf)
    z_neg = _comb(zero11, U, x)

    return (out, z_pos, z_neg)


# speedup vs baseline: 1.8284x; 1.3937x over previous
import functools
import math

import numpy as np
import jax
import jax.numpy as jnp
from jax import lax
from jax.experimental import pallas as pl
from jax.experimental.pallas import tpu as pltpu
from jax.experimental.pallas import tpu_sc as plsc

K = 10
N = 10000
E = 320000
F = 128
NC, NS, L = 2, 16, 16
NW = NC * NS
EPW = E // NW
CH = 80
NCH = EPW // CH
NRCH = N // CH
RK = -(-NRCH // NS)
EPT = E // NS
WCH = 2000

_mesh = plsc.VectorSubcoreMesh(core_axis_name="c", subcore_axis_name="s")


def _bernstein_monomial_matrix():
    C = np.zeros((K + 1, K + 1), dtype=np.float64)
    for j in range(K + 1):
        for p in range(j + 1):
            for q in range(K - j + 1):
                C[p + q, j] += (
                    math.comb(j, p) * ((-1) ** p) * math.comb(K - j, q) * math.comb(K, j)
                )
    return C / float(2 ** K)


_CMAT = _bernstein_monomial_matrix().astype(np.float32)


@functools.partial(
    pl.kernel,
    out_type=jax.ShapeDtypeStruct((E,), jnp.float32),
    mesh=_mesh,
    compiler_params=pltpu.CompilerParams(needs_layout_passes=False),
    scratch_types=[
        pltpu.VMEM((N,), jnp.float32),
        pltpu.VMEM((WCH,), jnp.int32),
        pltpu.VMEM((WCH,), jnp.int32),
        pltpu.VMEM((WCH,), jnp.float32),
        pltpu.VMEM((WCH,), jnp.float32),
        pltpu.VMEM((N,), jnp.float32),
        pltpu.VMEM((N,), jnp.float32),
        pltpu.VMEM((N,), jnp.float32),
        pltpu.VMEM_SHARED((NS, N), jnp.float32),
        pltpu.VMEM_SHARED((N,), jnp.float32),
    ],
)
def _norm_kernel(rows_hbm, cols_hbm, vals_hbm, w_hbm,
                 degloc, rbuf, cbuf, vbuf, wchunk, acc, tmp, disloc,
                 deg_all, dis_sh):
    c = lax.axis_index("c")
    t = lax.axis_index("s")

    @pl.when(c == 0)
    def _core0():
        def zloc(i, _):
            degloc[pl.ds(i * L, L)] = jnp.zeros((L,), jnp.float32)
            return 0
        lax.fori_loop(0, N // L, zloc, 0)

        base = t * EPT

        def dchunk(i, _):
            off = base + i * WCH
            pltpu.sync_copy(rows_hbm.at[pl.ds(off, WCH)], rbuf)
            pltpu.sync_copy(vals_hbm.at[pl.ds(off, WCH)], vbuf)

            def inner(k2, _2):
                r16 = rbuf[pl.ds(k2 * L, L)]
                v16 = vbuf[pl.ds(k2 * L, L)]
                plsc.addupdate_scatter(degloc, [r16], v16)
                return 0
            lax.fori_loop(0, WCH // L, inner, 0)
            return 0
        lax.fori_loop(0, EPT // WCH, dchunk, 0)

        pltpu.sync_copy(degloc, deg_all.at[t])
        plsc.subcore_barrier()

        @pl.when(t == 0)
        def _reduce():
            def zacc(i, _):
                acc[pl.ds(i * L, L)] = jnp.zeros((L,), jnp.float32)
                return 0
            lax.fori_loop(0, N // L, zacc, 0)

            def red(j, _):
                pltpu.sync_copy(deg_all.at[j], tmp)

                def addv(i, _2):
                    acc[pl.ds(i * L, L)] = acc[pl.ds(i * L, L)] + tmp[pl.ds(i * L, L)]
                    return 0
                lax.fori_loop(0, N // L, addv, 0)
                return 0
            lax.fori_loop(0, NS, red, 0)

            def rsq(i, _):
                d = acc[pl.ds(i * L, L)]
                bi = plsc.bitcast(d, jnp.int32)
                y = plsc.bitcast(jnp.int32(0x5F3759DF) - (bi >> 1), jnp.float32)
                for _it in range(4):
                    y = y * (1.5 - 0.5 * d * y * y)
                y = jnp.where(d > 0.0, y, jnp.zeros((L,), jnp.float32))
                acc[pl.ds(i * L, L)] = y
                return 0
            lax.fori_loop(0, N // L, rsq, 0)
            pltpu.sync_copy(acc, dis_sh)

        plsc.subcore_barrier()
        pltpu.sync_copy(dis_sh, disloc)

        def wloop(i, _):
            off = base + i * WCH
            pltpu.sync_copy(rows_hbm.at[pl.ds(off, WCH)], rbuf)
            pltpu.sync_copy(cols_hbm.at[pl.ds(off, WCH)], cbuf)
            pltpu.sync_copy(vals_hbm.at[pl.ds(off, WCH)], vbuf)

            def winner(k2, _2):
                r16 = rbuf[pl.ds(k2 * L, L)]
                c16 = cbuf[pl.ds(k2 * L, L)]
                dr = plsc.load_gather(disloc, [r16])
                dc = plsc.load_gather(disloc, [c16])
                wchunk[pl.ds(k2 * L, L)] = vbuf[pl.ds(k2 * L, L)] * dr * dc
                return 0
            lax.fori_loop(0, WCH // L, winner, 0)
            pltpu.sync_copy(wchunk, w_hbm.at[pl.ds(off, WCH)])
            return 0
        lax.fori_loop(0, EPT // WCH, wloop, 0)


RCH = 80


def _make_spmm(use_shuf, ch, epw):
    nch = epw // ch

    scratch = [
        pltpu.VMEM_SHARED((N, F), jnp.float32),
        pltpu.VMEM((ch, F), jnp.float32),
        pltpu.VMEM((ch, F), jnp.float32),
        pltpu.VMEM((ch,), jnp.int32),
        pltpu.VMEM((ch,), jnp.int32),
        pltpu.VMEM((ch,), jnp.float32),
        pltpu.VMEM((ch,), jnp.float32),
        pltpu.VMEM((epw,), jnp.int32),
        pltpu.VMEM((L,), jnp.float32),
    ]
    if use_shuf:
        scratch += [
            pltpu.VMEM((ch,), jnp.int32),
            pltpu.VMEM((ch,), jnp.int32),
            pltpu.VMEM((N,), jnp.int32),
        ]
    scratch += [
        pltpu.SemaphoreType.DMA,
        pltpu.SemaphoreType.DMA,
        pltpu.SemaphoreType.DMA,
        pltpu.SemaphoreType.DMA,
    ]

    @functools.partial(
        pl.kernel,
        out_type=jax.ShapeDtypeStruct((NC * N, F), jnp.float32),
        mesh=_mesh,
        compiler_params=pltpu.CompilerParams(needs_layout_passes=False),
        scratch_types=scratch,
    )
    def _spmm(v_hbm, rows_hbm, cols_hbm, w_hbm, s_hbm, shuf_hbm, u_hbm, *scr):
        if use_shuf:
            (u_sh, gbufA, gbufB, rowA, rowB, wA, wB, colall, sbuf,
             gidxA, gidxB, shufall, semGA, semGB, semEA, semEB) = scr
        else:
            (u_sh, gbufA, gbufB, rowA, rowB, wA, wB, colall, sbuf,
             semGA, semGB, semEA, semEB) = scr
            gidxA = gidxB = shufall = None

        c = lax.axis_index("c")
        t = lax.axis_index("s")
        wid = c * NS + t
        base = wid * epw

        gsta = gbufA.at[pl.ds(0, RCH)]

        def zb(i, _):
            for d in range(F // L):
                gbufA[i, pl.ds(d * L, L)] = jnp.zeros((L,), jnp.float32)
            return 0
        lax.fori_loop(0, RCH, zb, 0)
        for k in range(RK):
            idx = t + k * NS

            @pl.when(idx < NRCH)
            def _zchunk():
                pltpu.sync_copy(gsta, u_sh.at[pl.ds(idx * RCH, RCH)])

        pltpu.sync_copy(cols_hbm.at[pl.ds(base, epw)], colall)
        pltpu.sync_copy(s_hbm, sbuf)
        if use_shuf:
            pltpu.sync_copy(shuf_hbm, shufall)
        plsc.subcore_barrier()

        def gsrc(off, gidx):
            if use_shuf:
                return v_hbm.at[gidx]
            return v_hbm.at[colall.at[pl.ds(off, ch)]]

        def start(i, gbuf, rowbuf, wbuf, gidx, semG, semE):
            off = i * ch
            pltpu.async_copy(rows_hbm.at[pl.ds(base + off, ch)], rowbuf, semE)
            pltpu.async_copy(w_hbm.at[pl.ds(base + off, ch)], wbuf, semE)
            if use_shuf:
                for k in range(ch // L):
                    c16 = colall[pl.ds(off + k * L, L)]
                    gidx[pl.ds(k * L, L)] = plsc.load_gather(shufall, [c16])
            pltpu.async_copy(gsrc(off, gidx), gbuf, semG)

        def process(i, gbuf, rowbuf, wbuf, gidx, semG, semE):
            off = i * ch
            pltpu.make_async_copy(rows_hbm.at[pl.ds(base + off, ch)], rowbuf, semE).wait()
            pltpu.make_async_copy(w_hbm.at[pl.ds(base + off, ch)], wbuf, semE).wait()
            pltpu.make_async_copy(gsrc(off, gidx), gbuf, semG).wait()
            sv = sbuf[...]

            def scale(g, _2):
                w16 = wbuf[pl.ds(g * L, L)] * sv
                for j in range(L):
                    e = g * L + j
                    ws = w16[j]
                    for d in range(F // L):
                        gbuf[e, pl.ds(d * L, L)] = gbuf[e, pl.ds(d * L, L)] * ws
                return 0
            lax.fori_loop(0, ch // L, scale, 0)
            pltpu.sync_copy(gbuf, u_sh.at[rowbuf], add=True)

        start(0, gbufA, rowA, wA, gidxA, semGA, semEA)

        def pair(i2, _):
            i = i2 * 2
            start(i + 1, gbufB, rowB, wB, gidxB, semGB, semEB)
            process(i, gbufA, rowA, wA, gidxA, semGA, semEA)

            @pl.when(i + 2 < nch)
            def _startA():
                start(i + 2, gbufA, rowA, wA, gidxA, semGA, semEA)
            process(i + 1, gbufB, rowB, wB, gidxB, semGB, semEB)
            return 0
        lax.fori_loop(0, nch // 2, pair, 0)
        if nch % 2 == 1:
            process(nch - 1, gbufA, rowA, wA, gidxA, semGA, semEA)
        plsc.subcore_barrier()

        for k in range(RK):
            idx = t + k * NS

            @pl.when(idx < NRCH)
            def _dchunk():
                r0 = idx * RCH
                pltpu.sync_copy(u_sh.at[pl.ds(r0, RCH)], gsta)
                pltpu.sync_copy(gsta, u_hbm.at[pl.ds(c * N + r0, RCH)])

    return _spmm


CHP = 80
EPWP = -(-EPW // CHP) * CHP
EP2 = NW * EPWP
_spmm_plain = _make_spmm(False, CHP, EPWP)
_spmm_shuf = _make_spmm(True, CH, EPW)


def _comb_body(a_ref, u0_ref, u1_ref, x_ref, y_ref):
    y_ref[...] = u0_ref[...] + u1_ref[...] + a_ref[0, 0] * x_ref[...]


_comb_call = pl.pallas_call(
    _comb_body,
    grid=(10,),
    in_specs=[
        pl.BlockSpec(memory_space=pltpu.SMEM),
        pl.BlockSpec((N // 10, F), lambda i: (i, 0)),
        pl.BlockSpec((N // 10, F), lambda i: (i, 0)),
        pl.BlockSpec((N // 10, F), lambda i: (i, 0)),
    ],
    out_specs=pl.BlockSpec((N // 10, F), lambda i: (i, 0)),
    out_shape=jax.ShapeDtypeStruct((N, F), jnp.float32),
)


def _comb(alpha, U, x):
    return _comb_call(alpha.reshape(1, 1), U[:N], U[N:], x)


def _pad_edges(r, cc, ww):
    pad = EP2 - E
    rp = jnp.concatenate([r, jnp.zeros((pad,), jnp.int32)])
    cp = jnp.concatenate([cc, jnp.zeros((pad,), jnp.int32)])
    wp = jnp.concatenate([ww, jnp.zeros((pad,), jnp.float32)])
    return rp, cp, wp


def kernel(x, shuf, adj_indices, adj_values, neighbor_indices, neighbor_values, temp):
    TEMP = jax.nn.relu(temp)
    a = jnp.asarray(_CMAT) @ TEMP

    arow, acol = adj_indices[0], adj_indices[1]
    nrow, ncol = neighbor_indices[0], neighbor_indices[1]

    w = _norm_kernel(arow, acol, adj_values)

    arow_p, acol_p, w_p = _pad_edges(arow, acol, w)
    nrow_p, ncol_p, nv_p = _pad_edges(nrow, ncol, neighbor_values)

    ones16 = jnp.ones((L,), jnp.float32)
    zero11 = jnp.zeros((1, 1), jnp.float32)

    U = _spmm_plain(x, arow_p, acol_p, w_p, jnp.full((L,), a[K], jnp.float32), shuf)
    y = _comb(a[K - 1], U, x)
    for m in range(K - 2, -1, -1):
        U = _spmm_plain(y, arow_p, acol_p, w_p, ones16, shuf)
        y = _comb(a[m], U, x)
    out = y

    U = _spmm_plain(out, nrow_p, ncol_p, nv_p, ones16, shuf)
    z1 = _comb(zero11, U, x)
    U = _spmm_plain(z1, nrow_p, ncol_p, nv_p, ones16, shuf)
    z_pos = _comb(zero11, U, x)

    U = _spmm_shuf(out, nrow, ncol, neighbor_values, ones16, shuf)
    s1 = _comb(zero11, U, x)
    U = _spmm_plain(s1, nrow_p, ncol_p, nv_p, ones16, shuf)
    z_neg = _comb(zero11, U, x)

    return (out, z_pos, z_neg)
